# baseline (speedup 1.0000x reference)
import jax
import jax.numpy as jnp
from jax.experimental import pallas as pl
from jax.experimental.pallas import tpu as pltpu

_N = 10000
_TN = 400


def _leaky(v):
    return jnp.where(v >= 0, v, 0.01 * v)


def _tc1_body(feat_ref, w1_ref, so_ref, h1_ref):
    p = jnp.dot(feat_ref[...], w1_ref[...], preferred_element_type=jnp.float32)
    h1_ref[...] = p * so_ref[...]


def _tc1(features, w1, so_col):
    return pl.pallas_call(
        _tc1_body,
        grid=(_N // _TN,),
        in_specs=[
            pl.BlockSpec((_TN, 128), lambda i: (i, 0)),
            pl.BlockSpec((128, 5), lambda i: (0, 0)),
            pl.BlockSpec((_TN, 1), lambda i: (i, 0)),
        ],
        out_specs=pl.BlockSpec((_TN, 5), lambda i: (i, 0)),
        out_shape=jax.ShapeDtypeStruct((_N, 5), jnp.float32),
    )(features, w1, so_col)


def _tc2_body(r2_ref, si_ref, so_ref, w2_ref, w3_ref, h3_ref):
    q = r2_ref[...] * si_ref[...]
    z = q[:, 0:1] * w2_ref[0:1, :]
    for k in range(1, 5):
        z = z + q[:, k : k + 1] * w2_ref[k : k + 1, :]
    z = _leaky(z)
    h3_ref[...] = (
        jnp.dot(z, w3_ref[...], preferred_element_type=jnp.float32) * so_ref[...]
    )


def _tc2(r2, si_col, so_col, w2, w3):
    return pl.pallas_call(
        _tc2_body,
        grid=(_N // _TN,),
        in_specs=[
            pl.BlockSpec((_TN, 5), lambda i: (i, 0)),
            pl.BlockSpec((_TN, 1), lambda i: (i, 0)),
            pl.BlockSpec((_TN, 1), lambda i: (i, 0)),
            pl.BlockSpec((5, 4096), lambda i: (0, 0)),
            pl.BlockSpec((4096, 64), lambda i: (0, 0)),
        ],
        out_specs=pl.BlockSpec((_TN, 64), lambda i: (i, 0)),
        out_shape=jax.ShapeDtypeStruct((_N, 64), jnp.float32),
    )(r2, si_col, so_col, w2, w3)


def _tc3_body(
    r3_ref, si_ref, b3_ref, w1_ref, b1_ref, w2_ref, b2_ref, w3_ref, b3b_ref,
    w4_ref, b4_ref, w5_ref, b5_ref, out_ref,
):
    x = _leaky(r3_ref[...] * si_ref[...] + b3_ref[...])
    x = _leaky(jnp.dot(x, w1_ref[...], preferred_element_type=jnp.float32) + b1_ref[...])
    x = _leaky(jnp.dot(x, w2_ref[...], preferred_element_type=jnp.float32) + b2_ref[...])
    x = _leaky(jnp.dot(x, w3_ref[...], preferred_element_type=jnp.float32) + b3b_ref[...])
    x = _leaky(jnp.dot(x, w4_ref[...], preferred_element_type=jnp.float32) + b4_ref[...])
    out_ref[...] = (
        jnp.dot(x, w5_ref[...], preferred_element_type=jnp.float32) + b5_ref[...]
    )


def _tc3(r3, si_col, conv3_b, l1_W, l1_b, l2_W, l2_b, l3_W, l3_b, l4_W, l4_b, l5_W, l5_b):
    full = lambda shape: pl.BlockSpec(shape, lambda i: (0,) * len(shape))
    return pl.pallas_call(
        _tc3_body,
        grid=(_N // _TN,),
        in_specs=[
            pl.BlockSpec((_TN, 64), lambda i: (i, 0)),
            pl.BlockSpec((_TN, 1), lambda i: (i, 0)),
            full((1, 64)),
            full((64, 512)), full((1, 512)),
            full((512, 1024)), full((1, 1024)),
            full((1024, 512)), full((1, 512)),
            full((512, 64)), full((1, 64)),
            full((64, 1)), full((1, 1)),
        ],
        out_specs=pl.BlockSpec((_TN, 1), lambda i: (i, 0)),
        out_shape=jax.ShapeDtypeStruct((_N, 1), jnp.float32),
    )(
        r3, si_col, conv3_b.reshape(1, 64),
        l1_W, l1_b.reshape(1, 512),
        l2_W, l2_b.reshape(1, 1024),
        l3_W, l3_b.reshape(1, 512),
        l4_W, l4_b.reshape(1, 64),
        l5_W, l5_b.reshape(1, 1),
    )


def kernel(features, edge_index, weight, edge_weight, conv1_W, conv1_b,
           conv3_W, conv3_b, l1_W, l1_b, l2_W, l2_b, l3_W, l3_b, l4_W,
           l4_b, l5_W, l5_b):
    src = edge_index[0]
    dst = edge_index[1]
    ones = jnp.ones((src.shape[0],), dtype=jnp.float32)
    deg_out = jnp.clip(jax.ops.segment_sum(ones, src, num_segments=_N), 1.0, None)
    deg_in = jnp.clip(jax.ops.segment_sum(ones, dst, num_segments=_N), 1.0, None)
    so_col = (deg_out ** -0.5).reshape(_N, 1)
    si_col = (deg_in ** -0.5).reshape(_N, 1)

    h1 = _tc1(features, conv1_W, so_col)
    r1 = jax.ops.segment_sum(h1[src], dst, num_segments=_N)
    x1 = _leaky(r1 * si_col + conv1_b[None, :])

    h2 = x1 * so_col
    r2 = jax.ops.segment_sum(h2[src] * edge_weight[:, None], dst, num_segments=_N)

    h3 = _tc2(r2, si_col, so_col, weight, conv3_W)

    r3 = jax.ops.segment_sum(h3[src] * edge_weight[:, None], dst, num_segments=_N)

    return _tc3(r3, si_col, conv3_b, l1_W, l1_b, l2_W, l2_b, l3_W, l3_b,
                l4_W, l4_b, l5_W, l5_b)


# speedup vs baseline: 1.1395x; 1.1395x over previous
import functools

import jax
import jax.numpy as jnp
from jax import lax
from jax.experimental import pallas as pl
from jax.experimental.pallas import tpu as pltpu
from jax.experimental.pallas import tpu_sc as plsc

_N = 10000
_TN = 400

_NC, _NS, _L = 2, 16, 16
_NW = _NC * _NS
_NP = 10240
_E = 160000
_EPW = 5120
_EP = _NW * _EPW
_CS = _NP // _NS


def _deg_body(srcp, dstp, outs, outd, sbuf, dbuf, accs, accd, rbuf, res, sps, spd):
    c = lax.axis_index("c")
    s = lax.axis_index("s")
    wid = c * _NS + s
    base = wid * _EPW
    pltpu.sync_copy(srcp.at[pl.ds(base, _EPW)], sbuf)
    pltpu.sync_copy(dstp.at[pl.ds(base, _EPW)], dbuf)
    zero = jnp.zeros((_L,), jnp.float32)

    @pl.loop(0, _NP // _L)
    def _(i):
        accs[pl.ds(i * _L, _L)] = zero
        accd[pl.ds(i * _L, _L)] = zero

    lanes = lax.iota(jnp.int32, _L)

    @pl.loop(0, _EPW // _L)
    def _(i):
        eid = base + i * _L + lanes
        val = jnp.where(eid < _E, jnp.float32(1.0), jnp.float32(0.0))
        si = sbuf[pl.ds(i * _L, _L)]
        di = dbuf[pl.ds(i * _L, _L)]
        plsc.addupdate_scatter(accs, [si], val)
        plsc.addupdate_scatter(accd, [di], val)

    pltpu.sync_copy(accs, sps.at[s])
    pltpu.sync_copy(accd, spd.at[s])
    plsc.subcore_barrier()
    for sp, out in ((sps, outs), (spd, outd)):
        for k in range(_NS):
            pltpu.sync_copy(sp.at[k, pl.ds(s * _CS, _CS)], rbuf.at[k])

        @pl.loop(0, _CS // _L)
        def _(i):
            tot = rbuf[0, pl.ds(i * _L, _L)]
            for k in range(1, _NS):
                tot = tot + rbuf[k, pl.ds(i * _L, _L)]
            res[pl.ds(i * _L, _L)] = tot

        pltpu.sync_copy(res, out.at[c, pl.ds(s * _CS, _CS)])


@functools.cache
def _sc_degrees():
    mesh = plsc.VectorSubcoreMesh(
        core_axis_name="c", subcore_axis_name="s",
        num_cores=_NC, num_subcores=_NS)
    return pl.kernel(
        _deg_body,
        out_type=(jax.ShapeDtypeStruct((_NC, _NP), jnp.float32),
                  jax.ShapeDtypeStruct((_NC, _NP), jnp.float32)),
        mesh=mesh,
        compiler_params=pltpu.CompilerParams(needs_layout_passes=False),
        scratch_types=[
            pltpu.VMEM((_EPW,), jnp.int32),
            pltpu.VMEM((_EPW,), jnp.int32),
            pltpu.VMEM((_NP,), jnp.float32),
            pltpu.VMEM((_NP,), jnp.float32),
            pltpu.VMEM((_NS, _CS), jnp.float32),
            pltpu.VMEM((_CS,), jnp.float32),
            pltpu.VMEM_SHARED((_NS, _NP), jnp.float32),
            pltpu.VMEM_SHARED((_NS, _NP), jnp.float32),
        ],
    )


def _leaky(v):
    return jnp.where(v >= 0, v, 0.01 * v)


def _tc1_body(feat_ref, w1_ref, so_ref, h1_ref):
    h1_ref[...] = jnp.dot(feat_ref[...] * so_ref[...], w1_ref[...],
                          preferred_element_type=jnp.float32)


def _tc1(features, w1, so_col):
    return pl.pallas_call(
        _tc1_body,
        grid=(_N // _TN,),
        in_specs=[
            pl.BlockSpec((_TN, 128), lambda i: (i, 0)),
            pl.BlockSpec((128, 5), lambda i: (0, 0)),
            pl.BlockSpec((_TN, 1), lambda i: (i, 0)),
        ],
        out_specs=pl.BlockSpec((_TN, 5), lambda i: (i, 0)),
        out_shape=jax.ShapeDtypeStruct((_N, 5), jnp.float32),
    )(features, w1, so_col)


def _tc2_body(r2_ref, si_ref, so_ref, w2_ref, w3_ref, h3_ref):
    q = r2_ref[...] * si_ref[...]
    z = _leaky(jnp.dot(q, w2_ref[...], preferred_element_type=jnp.float32))
    h3_ref[...] = jnp.dot(z * so_ref[...], w3_ref[...],
                          preferred_element_type=jnp.float32)


def _tc2(r2, si_col, so_col, w2, w3):
    return pl.pallas_call(
        _tc2_body,
        grid=(_N // _TN,),
        in_specs=[
            pl.BlockSpec((_TN, 5), lambda i: (i, 0)),
            pl.BlockSpec((_TN, 1), lambda i: (i, 0)),
            pl.BlockSpec((_TN, 1), lambda i: (i, 0)),
            pl.BlockSpec((5, 4096), lambda i: (0, 0)),
            pl.BlockSpec((4096, 64), lambda i: (0, 0)),
        ],
        out_specs=pl.BlockSpec((_TN, 64), lambda i: (i, 0)),
        out_shape=jax.ShapeDtypeStruct((_N, 64), jnp.float32),
    )(r2, si_col, so_col, w2, w3)


def _tc3_body(
    r3_ref, si_ref, b3_ref, w1_ref, b1_ref, w2_ref, b2_ref, w3_ref, b3b_ref,
    w4_ref, b4_ref, w5_ref, b5_ref, out_ref,
):
    x = _leaky(r3_ref[...] * si_ref[...] + b3_ref[...])
    x = _leaky(jnp.dot(x, w1_ref[...], preferred_element_type=jnp.float32) + b1_ref[...])
    x = _leaky(jnp.dot(x, w2_ref[...], preferred_element_type=jnp.float32) + b2_ref[...])
    x = _leaky(jnp.dot(x, w3_ref[...], preferred_element_type=jnp.float32) + b3b_ref[...])
    x = _leaky(jnp.dot(x, w4_ref[...], preferred_element_type=jnp.float32) + b4_ref[...])
    out_ref[...] = (
        jnp.dot(x, w5_ref[...], preferred_element_type=jnp.float32) + b5_ref[...]
    )


def _tc3(r3, si_col, conv3_b, l1_W, l1_b, l2_W, l2_b, l3_W, l3_b, l4_W, l4_b, l5_W, l5_b):
    full = lambda shape: pl.BlockSpec(shape, lambda i: (0,) * len(shape))
    return pl.pallas_call(
        _tc3_body,
        grid=(_N // _TN,),
        in_specs=[
            pl.BlockSpec((_TN, 64), lambda i: (i, 0)),
            pl.BlockSpec((_TN, 1), lambda i: (i, 0)),
            full((1, 64)),
            full((64, 512)), full((1, 512)),
            full((512, 1024)), full((1, 1024)),
            full((1024, 512)), full((1, 512)),
            full((512, 64)), full((1, 64)),
            full((64, 1)), full((1, 1)),
        ],
        out_specs=pl.BlockSpec((_TN, 1), lambda i: (i, 0)),
        out_shape=jax.ShapeDtypeStruct((_N, 1), jnp.float32),
    )(
        r3, si_col, conv3_b.reshape(1, 64),
        l1_W, l1_b.reshape(1, 512),
        l2_W, l2_b.reshape(1, 1024),
        l3_W, l3_b.reshape(1, 512),
        l4_W, l4_b.reshape(1, 64),
        l5_W, l5_b.reshape(1, 1),
    )


def kernel(features, edge_index, weight, edge_weight, conv1_W, conv1_b,
           conv3_W, conv3_b, l1_W, l1_b, l2_W, l2_b, l3_W, l3_b, l4_W,
           l4_b, l5_W, l5_b):
    src = edge_index[0]
    dst = edge_index[1]
    srcp = jnp.pad(src, (0, _EP - _E))
    dstp = jnp.pad(dst, (0, _EP - _E))
    degs_p, degd_p = _sc_degrees()(srcp, dstp)
    deg_out = jnp.clip(degs_p[0, :_N] + degs_p[1, :_N], 1.0, None)
    deg_in = jnp.clip(degd_p[0, :_N] + degd_p[1, :_N], 1.0, None)
    so_col = (deg_out ** -0.5).reshape(_N, 1)
    si_col = (deg_in ** -0.5).reshape(_N, 1)

    h1 = _tc1(features, conv1_W, so_col)
    r1 = jax.ops.segment_sum(h1[src], dst, num_segments=_N)
    x1 = _leaky(r1 * si_col + conv1_b[None, :])

    h2 = x1 * so_col
    r2 = jax.ops.segment_sum(h2[src] * edge_weight[:, None], dst, num_segments=_N)

    h3 = _tc2(r2, si_col, so_col, weight, conv3_W)

    r3 = jax.ops.segment_sum(h3[src] * edge_weight[:, None], dst, num_segments=_N)

    return _tc3(r3, si_col, conv3_b, l1_W, l1_b, l2_W, l2_b, l3_W, l3_b,
                l4_W, l4_b, l5_W, l5_b)


# speedup vs baseline: 3.5840x; 3.1452x over previous
import functools

import jax
import jax.numpy as jnp
from jax import lax
from jax.experimental import pallas as pl
from jax.experimental.pallas import tpu as pltpu
from jax.experimental.pallas import tpu_sc as plsc

_N = 10000
_NP = 10240
_TN = 512

_NC, _NS, _L = 2, 16, 16
_NW = _NC * _NS
_E = 160000
_EPW = 5120
_EP = _NW * _EPW
_CS = _NP // _NS
_K = 128


def _leaky(v):
    return jnp.where(v >= 0, v, 0.01 * v)


def _sc_params():
    return pltpu.CompilerParams(needs_layout_passes=False,
                                use_tc_tiling_on_sc=False)


def _sc_mesh():
    return plsc.VectorSubcoreMesh(
        core_axis_name="c", subcore_axis_name="s",
        num_cores=_NC, num_subcores=_NS)


def _deg_body(srcp, dstp, outs, outd, sbuf, dbuf, accs, accd, rbuf, res, sps, spd):
    c = lax.axis_index("c")
    s = lax.axis_index("s")
    wid = c * _NS + s
    base = wid * _EPW
    pltpu.sync_copy(srcp.at[pl.ds(base, _EPW)], sbuf)
    pltpu.sync_copy(dstp.at[pl.ds(base, _EPW)], dbuf)
    zero = jnp.zeros((_L,), jnp.float32)

    @pl.loop(0, _NP // _L)
    def _(i):
        accs[pl.ds(i * _L, _L)] = zero
        accd[pl.ds(i * _L, _L)] = zero

    lanes = lax.iota(jnp.int32, _L)

    @pl.loop(0, _EPW // _L)
    def _(i):
        eid = base + i * _L + lanes
        val = jnp.where(eid < _E, jnp.float32(1.0), jnp.float32(0.0))
        si = sbuf[pl.ds(i * _L, _L)]
        di = dbuf[pl.ds(i * _L, _L)]
        plsc.addupdate_scatter(accs, [si], val)
        plsc.addupdate_scatter(accd, [di], val)

    pltpu.sync_copy(accs, sps.at[s])
    pltpu.sync_copy(accd, spd.at[s])
    plsc.subcore_barrier()
    for sp, out in ((sps, outs), (spd, outd)):
        for k in range(_NS):
            pltpu.sync_copy(sp.at[k, pl.ds(s * _CS, _CS)], rbuf.at[k])

        @pl.loop(0, _CS // _L)
        def _(i):
            tot = rbuf[0, pl.ds(i * _L, _L)]
            for k in range(1, _NS):
                tot = tot + rbuf[k, pl.ds(i * _L, _L)]
            res[pl.ds(i * _L, _L)] = tot

        pltpu.sync_copy(res, out.at[c, pl.ds(s * _CS, _CS)])


@functools.cache
def _sc_degrees():
    return pl.kernel(
        _deg_body,
        out_type=(jax.ShapeDtypeStruct((_NC, _NP), jnp.float32),
                  jax.ShapeDtypeStruct((_NC, _NP), jnp.float32)),
        mesh=_sc_mesh(),
        compiler_params=_sc_params(),
        scratch_types=[
            pltpu.VMEM((_EPW,), jnp.int32),
            pltpu.VMEM((_EPW,), jnp.int32),
            pltpu.VMEM((_NP,), jnp.float32),
            pltpu.VMEM((_NP,), jnp.float32),
            pltpu.VMEM((_NS, _CS), jnp.float32),
            pltpu.VMEM((_CS,), jnp.float32),
            pltpu.VMEM_SHARED((_NS, _NP), jnp.float32),
            pltpu.VMEM_SHARED((_NS, _NP), jnp.float32),
        ],
    )


def _mp_body(w, h, srcp, dstp, ewp, out, sidx, didx, ebuf, rows, sem, acc):
    nvr = w // _L
    c = lax.axis_index("c")
    s = lax.axis_index("s")
    wid = c * _NS + s
    base = wid * _EPW
    zero = jnp.zeros((_L,), jnp.float32)

    @pl.loop(0, _K)
    def _(e):
        for cc in range(nvr):
            rows[e, pl.ds(cc * _L, _L)] = zero

    for t in range(_CS // _K):
        pltpu.sync_copy(rows, acc.at[pl.ds(s * _CS + t * _K, _K), :])

    plsc.subcore_barrier()

    @pl.loop(0, _EPW // _K)
    def _(j):
        off = base + j * _K
        pltpu.sync_copy(srcp.at[pl.ds(off, _K)], sidx)
        pltpu.sync_copy(dstp.at[pl.ds(off, _K)], didx)
        pltpu.sync_copy(ewp.at[pl.ds(off, _K)], ebuf)
        pltpu.async_copy(h.at[sidx], rows, sem).wait()

        @pl.loop(0, _K // _L)
        def _(g):
            ewv = ebuf[pl.ds(g * _L, _L)]
            for e in range(_L):
                we = ewv[e]
                row = g * _L + e
                for cc in range(nvr):
                    sl = pl.ds(cc * _L, _L)
                    rows[row, sl] = rows[row, sl] * we

        pltpu.sync_copy(rows, acc.at[didx], add=True)

    plsc.subcore_barrier()
    pltpu.sync_copy(acc.at[pl.ds(s * _CS, _CS), :],
                    out.at[c, pl.ds(s * _CS, _CS), :])


@functools.cache
def _sc_mp(w):
    return pl.kernel(
        functools.partial(_mp_body, w),
        out_type=jax.ShapeDtypeStruct((_NC, _NP, w), jnp.float32),
        mesh=_sc_mesh(),
        compiler_params=_sc_params(),
        scratch_types=[
            pltpu.VMEM((_K,), jnp.int32),
            pltpu.VMEM((_K,), jnp.int32),
            pltpu.VMEM((_K,), jnp.float32),
            pltpu.VMEM((_K, w), jnp.float32),
            pltpu.SemaphoreType.DMA,
            pltpu.VMEM_SHARED((_NP, w), jnp.float32),
        ],
    )


def _tc1_body(feat_ref, w1_ref, so_ref, h1_ref):
    h1_ref[...] = jnp.dot(feat_ref[...] * so_ref[...], w1_ref[...],
                          preferred_element_type=jnp.float32)


def _tc1(features_p, w1p, so_col):
    return pl.pallas_call(
        _tc1_body,
        grid=(_NP // _TN,),
        in_specs=[
            pl.BlockSpec((_TN, 128), lambda i: (i, 0)),
            pl.BlockSpec((128, 16), lambda i: (0, 0)),
            pl.BlockSpec((_TN, 1), lambda i: (i, 0)),
        ],
        out_specs=pl.BlockSpec((_TN, 16), lambda i: (i, 0)),
        out_shape=jax.ShapeDtypeStruct((_NP, 16), jnp.float32),
    )(features_p, w1p, so_col)


def _tcmid_body(a_ref, b_ref, si_ref, so_ref, b1_ref, h2_ref):
    x1 = _leaky((a_ref[...] + b_ref[...]) * si_ref[...] + b1_ref[...])
    h2_ref[...] = x1 * so_ref[...]


def _tcmid(r1a, r1b, si_col, so_col, b1p):
    return pl.pallas_call(
        _tcmid_body,
        grid=(_NP // _TN,),
        in_specs=[
            pl.BlockSpec((_TN, 16), lambda i: (i, 0)),
            pl.BlockSpec((_TN, 16), lambda i: (i, 0)),
            pl.BlockSpec((_TN, 1), lambda i: (i, 0)),
            pl.BlockSpec((_TN, 1), lambda i: (i, 0)),
            pl.BlockSpec((1, 16), lambda i: (0, 0)),
        ],
        out_specs=pl.BlockSpec((_TN, 16), lambda i: (i, 0)),
        out_shape=jax.ShapeDtypeStruct((_NP, 16), jnp.float32),
    )(r1a, r1b, si_col, so_col, b1p)


def _tc2_body(a_ref, b_ref, si_ref, so_ref, w2_ref, w3_ref, h3_ref):
    q = (a_ref[...] + b_ref[...]) * si_ref[...]
    z = _leaky(jnp.dot(q, w2_ref[...], preferred_element_type=jnp.float32))
    h3_ref[...] = jnp.dot(z * so_ref[...], w3_ref[...],
                          preferred_element_type=jnp.float32)


def _tc2(r2a, r2b, si_col, so_col, w2p, w3):
    return pl.pallas_call(
        _tc2_body,
        grid=(_NP // _TN,),
        in_specs=[
            pl.BlockSpec((_TN, 16), lambda i: (i, 0)),
            pl.BlockSpec((_TN, 16), lambda i: (i, 0)),
            pl.BlockSpec((_TN, 1), lambda i: (i, 0)),
            pl.BlockSpec((_TN, 1), lambda i: (i, 0)),
            pl.BlockSpec((16, 4096), lambda i: (0, 0)),
            pl.BlockSpec((4096, 64), lambda i: (0, 0)),
        ],
        out_specs=pl.BlockSpec((_TN, 64), lambda i: (i, 0)),
        out_shape=jax.ShapeDtypeStruct((_NP, 64), jnp.float32),
    )(r2a, r2b, si_col, so_col, w2p, w3)


def _tc3_body(
    a_ref, b_ref, si_ref, b3_ref, w1_ref, b1_ref, w2_ref, b2_ref, w3_ref,
    b3b_ref, w4_ref, b4_ref, w5_ref, b5_ref, out_ref,
):
    x = _leaky((a_ref[...] + b_ref[...]) * si_ref[...] + b3_ref[...])
    x = _leaky(jnp.dot(x, w1_ref[...], preferred_element_type=jnp.float32) + b1_ref[...])
    x = _leaky(jnp.dot(x, w2_ref[...], preferred_element_type=jnp.float32) + b2_ref[...])
    x = _leaky(jnp.dot(x, w3_ref[...], preferred_element_type=jnp.float32) + b3b_ref[...])
    x = _leaky(jnp.dot(x, w4_ref[...], preferred_element_type=jnp.float32) + b4_ref[...])
    out_ref[...] = (
        jnp.dot(x, w5_ref[...], preferred_element_type=jnp.float32) + b5_ref[...]
    )


def _tc3(r3a, r3b, si_col, conv3_b, l1_W, l1_b, l2_W, l2_b, l3_W, l3_b,
         l4_W, l4_b, l5_W, l5_b):
    full = lambda shape: pl.BlockSpec(shape, lambda i: (0,) * len(shape))
    return pl.pallas_call(
        _tc3_body,
        grid=(_NP // _TN,),
        in_specs=[
            pl.BlockSpec((_TN, 64), lambda i: (i, 0)),
            pl.BlockSpec((_TN, 64), lambda i: (i, 0)),
            pl.BlockSpec((_TN, 1), lambda i: (i, 0)),
            full((1, 64)),
            full((64, 512)), full((1, 512)),
            full((512, 1024)), full((1, 1024)),
            full((1024, 512)), full((1, 512)),
            full((512, 64)), full((1, 64)),
            full((64, 1)), full((1, 1)),
        ],
        out_specs=pl.BlockSpec((_TN, 1), lambda i: (i, 0)),
        out_shape=jax.ShapeDtypeStruct((_NP, 1), jnp.float32),
    )(
        r3a, r3b, si_col, conv3_b.reshape(1, 64),
        l1_W, l1_b.reshape(1, 512),
        l2_W, l2_b.reshape(1, 1024),
        l3_W, l3_b.reshape(1, 512),
        l4_W, l4_b.reshape(1, 64),
        l5_W, l5_b.reshape(1, 1),
    )


def kernel(features, edge_index, weight, edge_weight, conv1_W, conv1_b,
           conv3_W, conv3_b, l1_W, l1_b, l2_W, l2_b, l3_W, l3_b, l4_W,
           l4_b, l5_W, l5_b):
    src = edge_index[0]
    dst = edge_index[1]
    srcp = jnp.pad(src, (0, _EP - _E))
    dstp = jnp.pad(dst, (0, _EP - _E))
    ewp = jnp.pad(edge_weight, (0, _EP - _E))
    onesp = jnp.pad(jnp.ones((_E,), jnp.float32), (0, _EP - _E))
    features_p = jnp.pad(features, ((0, _NP - _N), (0, 0)))
    w1p = jnp.pad(conv1_W, ((0, 0), (0, 11)))
    b1p = jnp.pad(conv1_b, (0, 11)).reshape(1, 16)
    w2p = jnp.pad(weight, ((0, 11), (0, 0)))

    degs_p, degd_p = _sc_degrees()(srcp, dstp)
    deg_out = jnp.clip(degs_p[0] + degs_p[1], 1.0, None)
    deg_in = jnp.clip(degd_p[0] + degd_p[1], 1.0, None)
    so_col = (deg_out ** -0.5).reshape(_NP, 1)
    si_col = (deg_in ** -0.5).reshape(_NP, 1)

    h1 = _tc1(features_p, w1p, so_col)
    r1p = _sc_mp(16)(h1, srcp, dstp, onesp)

    h2 = _tcmid(r1p[0], r1p[1], si_col, so_col, b1p)
    r2p = _sc_mp(16)(h2, srcp, dstp, ewp)

    h3 = _tc2(r2p[0], r2p[1], si_col, so_col, w2p, conv3_W)

    r3p = _sc_mp(64)(h3, srcp, dstp, ewp)

    out = _tc3(r3p[0], r3p[1], si_col, conv3_b, l1_W, l1_b, l2_W, l2_b,
               l3_W, l3_b, l4_W, l4_b, l5_W, l5_b)
    return out[:_N]
